# baseline (speedup 1.0000x reference)
import jax
import jax.numpy as jnp
from jax.experimental import pallas as pl
from jax.experimental.pallas import tpu as pltpu

N, D, E, H_R, H_E = 4096, 1024, 16, 64, 128
TBLK = 1024
HF = E * H_E
KX = HF + E


def _router_kernel(x_ref, rw1_ref, rb1_ref, rw2_ref, rb2_ref,
                   w_ref, wtop_ref):
    xb = x_ref[...]
    hr = jnp.maximum(
        jnp.dot(xb, rw1_ref[...], preferred_element_type=jnp.float32)
        + rb1_ref[...][None, :], 0.0)
    logits = (jnp.dot(hr, rw2_ref[...], preferred_element_type=jnp.float32)
              + rb2_ref[...][None, :])
    logits = logits - jnp.max(logits, axis=-1, keepdims=True)
    ew = jnp.exp(logits)
    w = ew / jnp.sum(ew, axis=-1, keepdims=True)
    w_ref[...] = w
    cols = jax.lax.broadcasted_iota(jnp.int32, w.shape, 1)
    i1 = jnp.argmax(w, axis=-1)[:, None]
    w2 = jnp.where(cols == i1, -jnp.inf, w)
    i2 = jnp.argmax(w2, axis=-1)[:, None]
    mask = (cols == i1) | (cols == i2)
    wt = jnp.where(mask, w, 0.0)
    wtop_ref[...] = wt / (jnp.sum(wt, axis=-1, keepdims=True) + 1e-8)


def _expert_kernel(x_ref, wtop_ref, ew1_ref, eb1_ref, ew2_ref,
                   y_ref, hs_ref):
    xb = x_ref[...].astype(jnp.bfloat16)
    pre = (jnp.dot(xb, ew1_ref[...], preferred_element_type=jnp.float32)
           + eb1_ref[...])
    h = jnp.tanh(pre)
    wt = wtop_ref[...]
    gates = jnp.tile(wt, (1, HF // E))
    hs_ref[:, :HF] = (h * gates).astype(jnp.bfloat16)
    hs_ref[:, HF:] = wt.astype(jnp.bfloat16)
    y_ref[...] = jnp.dot(hs_ref[...], ew2_ref[...],
                         preferred_element_type=jnp.float32)


@jax.jit
def kernel(x, rw1, rb1, rw2, rb2, ew1, eb1, ew2, eb2):
    w, wtop = pl.pallas_call(
        _router_kernel,
        grid=(1,),
        in_specs=[
            pl.BlockSpec((N, D), lambda i: (0, 0)),
            pl.BlockSpec((D, H_R), lambda i: (0, 0)),
            pl.BlockSpec((H_R,), lambda i: (0,)),
            pl.BlockSpec((H_R, E), lambda i: (0, 0)),
            pl.BlockSpec((E,), lambda i: (0,)),
        ],
        out_specs=[
            pl.BlockSpec((N, E), lambda i: (0, 0)),
            pl.BlockSpec((N, E), lambda i: (0, 0)),
        ],
        out_shape=[
            jax.ShapeDtypeStruct((N, E), jnp.float32),
            jax.ShapeDtypeStruct((N, E), jnp.float32),
        ],
    )(x, rw1, rb1, rw2, rb2)

    ew1f = ew1.transpose(1, 2, 0).reshape(D, HF).astype(jnp.bfloat16)
    eb1f = eb1.transpose(1, 0).reshape(1, HF)
    ew2f = jnp.concatenate(
        [ew2.transpose(1, 0, 2).reshape(HF, D), eb2],
        axis=0).astype(jnp.bfloat16)

    y = pl.pallas_call(
        _expert_kernel,
        grid=(N // TBLK,),
        in_specs=[
            pl.BlockSpec((TBLK, D), lambda i: (i, 0)),
            pl.BlockSpec((TBLK, E), lambda i: (i, 0)),
            pl.BlockSpec((D, HF), lambda i: (0, 0)),
            pl.BlockSpec((1, HF), lambda i: (0, 0)),
            pl.BlockSpec((KX, D), lambda i: (0, 0)),
        ],
        out_specs=pl.BlockSpec((TBLK, D), lambda i: (i, 0)),
        out_shape=jax.ShapeDtypeStruct((N, D), jnp.float32),
        scratch_shapes=[pltpu.VMEM((TBLK, KX), jnp.bfloat16)],
        compiler_params=pltpu.CompilerParams(
            dimension_semantics=("parallel",)),
    )(x, wtop, ew1f, eb1f, ew2f)
    return (y, w)


# speedup vs baseline: 1.0908x; 1.0908x over previous
import jax
import jax.numpy as jnp
from jax.experimental import pallas as pl
from jax.experimental.pallas import tpu as pltpu

N, D, E, H_R, H_E = 4096, 1024, 16, 64, 128
TBLK = 1024
HF = E * H_E
KX = HF + H_E
RT = N // E


def _prep_kernel(x_ref, rw1_ref, rb1_ref, rw2_ref, rb2_ref,
                 ew1_ref, ew2_ref, eb2_ref,
                 w_ref, wtop_ref, ew1f_ref, ew2f_ref):
    j = pl.program_id(0)

    @pl.when(j < E)
    def _router_and_w1():
        xb = x_ref[...]
        hr = jnp.maximum(
            jnp.dot(xb, rw1_ref[...], preferred_element_type=jnp.float32)
            + rb1_ref[...][None, :], 0.0)
        logits = (jnp.dot(hr, rw2_ref[...],
                          preferred_element_type=jnp.float32)
                  + rb2_ref[...][None, :])
        logits = logits - jnp.max(logits, axis=-1, keepdims=True)
        ew = jnp.exp(logits)
        w = ew / jnp.sum(ew, axis=-1, keepdims=True)
        w_ref[...] = w
        cols = jax.lax.broadcasted_iota(jnp.int32, w.shape, 1)
        i1 = jnp.argmax(w, axis=-1)[:, None]
        w2 = jnp.where(cols == i1, -jnp.inf, w)
        i2 = jnp.argmax(w2, axis=-1)[:, None]
        mask = (cols == i1) | (cols == i2)
        wt = jnp.where(mask, w, 0.0)
        wtop_ref[...] = wt / (jnp.sum(wt, axis=-1, keepdims=True) + 1e-8)
        ew1f_ref[...] = ew1_ref[0].astype(jnp.bfloat16)
        ew2f_ref[...] = ew2_ref[0].astype(jnp.bfloat16)

    @pl.when(j == E)
    def _eb2_pad_block():
        ew2f_ref[0:E, :] = eb2_ref[...].astype(jnp.bfloat16)
        ew2f_ref[E:H_E, :] = jnp.zeros((H_E - E, D), jnp.bfloat16)


def _expert_kernel(x_ref, wtop_ref, ew1f_ref, eb1f_ref, ew2f_ref,
                   y_ref, hs_ref):
    i = pl.program_id(0)

    @pl.when(i == 0)
    def _zero_pad_cols():
        hs_ref[:, HF + E:] = jnp.zeros((TBLK, KX - HF - E), jnp.bfloat16)

    xb = x_ref[...].astype(jnp.bfloat16)
    pre = (jnp.dot(xb, ew1f_ref[...], preferred_element_type=jnp.float32)
           + eb1f_ref[...])
    h = jnp.tanh(pre)
    wt = wtop_ref[...]
    gates = jnp.broadcast_to(wt[:, :, None], (TBLK, E, H_E)).reshape(TBLK, HF)
    hs_ref[:, :HF] = (h * gates).astype(jnp.bfloat16)
    hs_ref[:, HF:HF + E] = wt.astype(jnp.bfloat16)
    y_ref[...] = jnp.dot(hs_ref[...], ew2f_ref[...],
                         preferred_element_type=jnp.float32)


@jax.jit
def kernel(x, rw1, rb1, rw2, rb2, ew1, eb1, ew2, eb2):
    cl = lambda j: jnp.minimum(j, E - 1)
    w, wtop, ew1f, ew2f = pl.pallas_call(
        _prep_kernel,
        grid=(E + 1,),
        in_specs=[
            pl.BlockSpec((RT, D), lambda j: (cl(j), 0)),
            pl.BlockSpec((D, H_R), lambda j: (0, 0)),
            pl.BlockSpec((H_R,), lambda j: (0,)),
            pl.BlockSpec((H_R, E), lambda j: (0, 0)),
            pl.BlockSpec((E,), lambda j: (0,)),
            pl.BlockSpec((1, D, H_E), lambda j: (cl(j), 0, 0)),
            pl.BlockSpec((1, H_E, D), lambda j: (cl(j), 0, 0)),
            pl.BlockSpec((E, D), lambda j: (0, 0)),
        ],
        out_specs=[
            pl.BlockSpec((RT, E), lambda j: (cl(j), 0)),
            pl.BlockSpec((RT, E), lambda j: (cl(j), 0)),
            pl.BlockSpec((D, H_E), lambda j: (0, cl(j))),
            pl.BlockSpec((H_E, D), lambda j: (jnp.minimum(j, E), 0)),
        ],
        out_shape=[
            jax.ShapeDtypeStruct((N, E), jnp.float32),
            jax.ShapeDtypeStruct((N, E), jnp.float32),
            jax.ShapeDtypeStruct((D, HF), jnp.bfloat16),
            jax.ShapeDtypeStruct((KX, D), jnp.bfloat16),
        ],
    )(x, rw1, rb1, rw2, rb2, ew1, ew2, eb2)

    eb1f = eb1.reshape(1, HF)
    y = pl.pallas_call(
        _expert_kernel,
        grid=(N // TBLK,),
        in_specs=[
            pl.BlockSpec((TBLK, D), lambda i: (i, 0)),
            pl.BlockSpec((TBLK, E), lambda i: (i, 0)),
            pl.BlockSpec((D, HF), lambda i: (0, 0)),
            pl.BlockSpec((1, HF), lambda i: (0, 0)),
            pl.BlockSpec((KX, D), lambda i: (0, 0)),
        ],
        out_specs=pl.BlockSpec((TBLK, D), lambda i: (i, 0)),
        out_shape=jax.ShapeDtypeStruct((N, D), jnp.float32),
        scratch_shapes=[pltpu.VMEM((TBLK, KX), jnp.bfloat16)],
        compiler_params=pltpu.CompilerParams(
            dimension_semantics=("arbitrary",)),
    )(x, wtop, ew1f, eb1f, ew2f)
    return (y, w)


# speedup vs baseline: 1.1225x; 1.0291x over previous
import jax
import jax.numpy as jnp
from jax.experimental import pallas as pl
from jax.experimental.pallas import tpu as pltpu

N, D, E, H_R, H_E = 4096, 1024, 16, 64, 128
TBLK = 1024
HF = E * H_E
KX = HF + E


def _router_kernel(x_ref, rw1_ref, rb1_ref, rw2_ref, rb2_ref,
                   w_ref, wtop_ref):
    xb = x_ref[...]
    hr = jnp.maximum(
        jnp.dot(xb, rw1_ref[...], preferred_element_type=jnp.float32)
        + rb1_ref[...][None, :], 0.0)
    logits = (jnp.dot(hr, rw2_ref[...], preferred_element_type=jnp.float32)
              + rb2_ref[...][None, :])
    logits = logits - jnp.max(logits, axis=-1, keepdims=True)
    ew = jnp.exp(logits)
    w = ew / jnp.sum(ew, axis=-1, keepdims=True)
    w_ref[...] = w
    cols = jax.lax.broadcasted_iota(jnp.int32, w.shape, 1)
    i1 = jnp.argmax(w, axis=-1)[:, None]
    w2 = jnp.where(cols == i1, -jnp.inf, w)
    i2 = jnp.argmax(w2, axis=-1)[:, None]
    mask = (cols == i1) | (cols == i2)
    wt = jnp.where(mask, w, 0.0)
    wtop_ref[...] = wt / (jnp.sum(wt, axis=-1, keepdims=True) + 1e-8)


NCH = 2
CH = TBLK // NCH


def _expert_kernel(x_ref, wtop_ref, ew1_ref, eb1_ref, ew2_ref,
                   y_ref, hs_ref):
    for c in range(NCH):
        r = slice(c * CH, (c + 1) * CH)
        xb = x_ref[r, :].astype(jnp.bfloat16)
        pre = (jnp.dot(xb, ew1_ref[...], preferred_element_type=jnp.float32)
               + eb1_ref[...])
        h = jnp.tanh(pre)
        wt = wtop_ref[r, :]
        gates = jnp.broadcast_to(
            wt[:, :, None], (CH, E, H_E)).reshape(CH, HF)
        hs_ref[r, :HF] = (h * gates).astype(jnp.bfloat16)
        hs_ref[r, HF:] = wt.astype(jnp.bfloat16)
    for c in range(NCH):
        r = slice(c * CH, (c + 1) * CH)
        y_ref[r, :] = jnp.dot(hs_ref[r, :], ew2_ref[...],
                              preferred_element_type=jnp.float32)


@jax.jit
def kernel(x, rw1, rb1, rw2, rb2, ew1, eb1, ew2, eb2):
    w, wtop = pl.pallas_call(
        _router_kernel,
        grid=(1,),
        in_specs=[
            pl.BlockSpec((N, D), lambda i: (0, 0)),
            pl.BlockSpec((D, H_R), lambda i: (0, 0)),
            pl.BlockSpec((H_R,), lambda i: (0,)),
            pl.BlockSpec((H_R, E), lambda i: (0, 0)),
            pl.BlockSpec((E,), lambda i: (0,)),
        ],
        out_specs=[
            pl.BlockSpec((N, E), lambda i: (0, 0)),
            pl.BlockSpec((N, E), lambda i: (0, 0)),
        ],
        out_shape=[
            jax.ShapeDtypeStruct((N, E), jnp.float32),
            jax.ShapeDtypeStruct((N, E), jnp.float32),
        ],
    )(x, rw1, rb1, rw2, rb2)

    ew1f = ew1.transpose(1, 0, 2).reshape(D, HF).astype(jnp.bfloat16)
    eb1f = eb1.reshape(1, HF)
    ew2f = jnp.concatenate(
        [ew2.reshape(HF, D), eb2], axis=0).astype(jnp.bfloat16)

    y = pl.pallas_call(
        _expert_kernel,
        grid=(N // TBLK,),
        in_specs=[
            pl.BlockSpec((TBLK, D), lambda i: (i, 0)),
            pl.BlockSpec((TBLK, E), lambda i: (i, 0)),
            pl.BlockSpec((D, HF), lambda i: (0, 0)),
            pl.BlockSpec((1, HF), lambda i: (0, 0)),
            pl.BlockSpec((KX, D), lambda i: (0, 0)),
        ],
        out_specs=pl.BlockSpec((TBLK, D), lambda i: (i, 0)),
        out_shape=jax.ShapeDtypeStruct((N, D), jnp.float32),
        scratch_shapes=[pltpu.VMEM((TBLK, KX), jnp.bfloat16)],
        compiler_params=pltpu.CompilerParams(
            dimension_semantics=("parallel",)),
    )(x, wtop, ew1f, eb1f, ew2f)
    return (y, w)
